# baseline (speedup 1.0000x reference)
import functools

import jax
import jax.numpy as jnp
from jax.experimental import pallas as pl
from jax.experimental.pallas import tpu as pltpu

_NPOINTS = (1024, 512, 128)
_RADII = (0.05, 0.1, 0.2)
_NSAMPLES = (32, 32, 64)


def _sqdist(src, dst):
    return jnp.sum((src[:, :, None, :] - dst[:, None, :, :]) ** 2, axis=-1)


def _index_points(points, idx):
    B = points.shape[0]
    batch = jnp.arange(B).reshape((B,) + (1,) * (idx.ndim - 1))
    return points[batch, idx]


def _fps(xyz, npoint):
    B, N, _ = xyz.shape

    def body(i, carry):
        centroids, distance, farthest = carry
        centroids = centroids.at[:, i].set(farthest)
        centroid = xyz[jnp.arange(B), farthest][:, None, :]
        dist = jnp.sum((xyz - centroid) ** 2, axis=-1)
        distance = jnp.minimum(distance, dist)
        farthest = jnp.argmax(distance, axis=-1).astype(jnp.int32)
        return centroids, distance, farthest

    centroids = jnp.zeros((B, npoint), dtype=jnp.int32)
    distance = jnp.full((B, N), 1e10, dtype=jnp.float32)
    farthest = jnp.zeros((B,), dtype=jnp.int32)
    centroids, _, _ = jax.lax.fori_loop(0, npoint, body,
                                        (centroids, distance, farthest))
    return centroids


def _ball_query(radius, nsample, xyz, new_xyz):
    B, N, _ = xyz.shape
    S = new_xyz.shape[1]
    sqrdists = _sqdist(new_xyz, xyz)
    group_idx = jnp.broadcast_to(jnp.arange(N, dtype=jnp.int32), (B, S, N))
    group_idx = jnp.where(sqrdists > radius ** 2, N, group_idx)
    group_idx = jnp.sort(group_idx, axis=-1)[:, :, :nsample]
    group_first = jnp.broadcast_to(group_idx[:, :, :1], (B, S, nsample))
    group_idx = jnp.where(group_idx == N, group_first, group_idx)
    return group_idx


def _mlp_bn(h, layers):
    for layer in layers:
        h = jnp.einsum('oc,bckn->bokn', layer['W'], h) \
            + layer['b'][None, :, None, None]
        mean = jnp.mean(h, axis=(0, 2, 3), keepdims=True)
        var = jnp.var(h, axis=(0, 2, 3), keepdims=True)
        h = (h - mean) / jnp.sqrt(var + 1e-5)
        h = h * layer['gamma'][None, :, None, None] \
            + layer['beta'][None, :, None, None]
        h = jax.nn.relu(h)
    return h


def _sa(xyz_ch, points_ch, layers, npoint, radius, nsample, group_all):
    xyz = jnp.transpose(xyz_ch, (0, 2, 1))
    points = None if points_ch is None else jnp.transpose(points_ch, (0, 2, 1))
    if group_all:
        B, N, C = xyz.shape
        new_xyz = jnp.zeros((B, 1, C), dtype=xyz.dtype)
        grouped_xyz = xyz[:, None, :, :]
        new_points = jnp.concatenate([grouped_xyz, points[:, None, :, :]],
                                     axis=-1)
    else:
        fps_idx = _fps(xyz, npoint)
        new_xyz = _index_points(xyz, fps_idx)
        idx = _ball_query(radius, nsample, xyz, new_xyz)
        grouped_xyz = _index_points(xyz, idx) - new_xyz[:, :, None, :]
        if points is not None:
            grouped_points = _index_points(points, idx)
            new_points = jnp.concatenate([grouped_xyz, grouped_points], axis=-1)
        else:
            new_points = grouped_xyz
    h = jnp.transpose(new_points, (0, 3, 2, 1))
    h = _mlp_bn(h, layers)
    new_points_out = jnp.max(h, axis=2)
    new_xyz_out = jnp.transpose(new_xyz, (0, 2, 1))
    return new_xyz_out, new_points_out


def _copy_kernel(x_ref, o_ref):
    o_ref[...] = x_ref[...]


def kernel(x, params):
    l1_xyz, l1_points = _sa(x, None, params[0], _NPOINTS[0], _RADII[0],
                            _NSAMPLES[0], False)
    l2_xyz, l2_points = _sa(l1_xyz, l1_points, params[1], _NPOINTS[1],
                            _RADII[1], _NSAMPLES[1], False)
    l3_xyz, l3_points = _sa(l2_xyz, l2_points, params[2], _NPOINTS[2],
                            _RADII[2], _NSAMPLES[2], False)
    _, l4_points = _sa(l3_xyz, l3_points, params[3], None, None, None, True)
    out = jnp.squeeze(l4_points, axis=-1)
    out = pl.pallas_call(
        _copy_kernel,
        out_shape=jax.ShapeDtypeStruct(out.shape, out.dtype),
    )(out)
    return out


# speedup vs baseline: 10.4517x; 10.4517x over previous
import functools

import jax
import jax.numpy as jnp
from jax import lax
from jax.experimental import pallas as pl
from jax.experimental.pallas import tpu as pltpu
from jax.experimental.pallas import tpu_sc as plsc

_B = 8
_N0 = 2048
_NPOINTS = (1024, 512, 128)
_RADII = (0.05, 0.1, 0.2)
_NSAMPLES = (32, 32, 64)
_PAD = 16


def _fps_levels_kernel(xs, ys, zs, ox1, oy1, oz1, ox2, oy2, oz2,
                       ox3, oy3, oz3):
    def run_level(src_x, src_y, src_z, N, S, ox, oy, oz):
        iota = lax.broadcasted_iota(jnp.int32, (_B, N), 1)
        sel_iota = lax.broadcasted_iota(jnp.int32, (_B, S), 1)
        ox[...] = jnp.zeros((_B, S), jnp.float32)
        oy[...] = jnp.zeros((_B, S), jnp.float32)
        oz[...] = jnp.zeros((_B, S), jnp.float32)

        def body(i, carry):
            far, dist = carry
            oh = (iota == far).astype(jnp.float32)
            cx = jnp.sum(src_x * oh, axis=1, keepdims=True)
            cy = jnp.sum(src_y * oh, axis=1, keepdims=True)
            cz = jnp.sum(src_z * oh, axis=1, keepdims=True)
            soh = (sel_iota == i).astype(jnp.float32)
            ox[...] += cx * soh
            oy[...] += cy * soh
            oz[...] += cz * soh
            dx = src_x - cx
            dy = src_y - cy
            dz = src_z - cz
            d = dx * dx + dy * dy + dz * dz
            dist = jnp.minimum(dist, d)
            m = jnp.max(dist, axis=1, keepdims=True)
            far = jnp.min(jnp.where(dist == m, iota, N), axis=1,
                          keepdims=True).astype(jnp.int32)
            return far, dist

        far0 = jnp.zeros((_B, 1), jnp.int32)
        dist0 = jnp.full((_B, N), 1e10, jnp.float32)
        lax.fori_loop(0, S, body, (far0, dist0))

    run_level(xs[...], ys[...], zs[...], _N0, _NPOINTS[0], ox1, oy1, oz1)
    run_level(ox1[...], oy1[...], oz1[...], _NPOINTS[0], _NPOINTS[1],
              ox2, oy2, oz2)
    run_level(ox2[...], oy2[...], oz2[...], _NPOINTS[1], _NPOINTS[2],
              ox3, oy3, oz3)


def _run_fps(xs, ys, zs):
    shapes = []
    for S in _NPOINTS:
        shapes += [jax.ShapeDtypeStruct((_B, S), jnp.float32)] * 3
    return pl.pallas_call(
        _fps_levels_kernel,
        out_shape=shapes,
    )(xs, ys, zs)


def _ball_kernel(px, py, pz, cx, cy, cz, o_ref, *, r2, k, N, TS):
    pxv = px[0]
    pyv = py[0]
    pzv = pz[0]
    cxv = cx[0]
    cyv = cy[0]
    czv = cz[0]
    dx = cxv - pxv
    dy = cyv - pyv
    dz = czv - pzv
    d = dx * dx + dy * dy + dz * dz
    mask = jnp.logical_not(d > r2)
    cnt = mask.astype(jnp.int32)
    sh = 1
    while sh < N:
        cnt = cnt + jnp.concatenate(
            [jnp.zeros((TS, sh), jnp.int32), cnt[:, :N - sh]], axis=1)
        sh *= 2
    total = cnt[:, N - 1:N]
    kiota = lax.broadcasted_iota(jnp.int32, (TS, k), 1)
    acc = jnp.zeros((TS, k), jnp.int32)
    pos1 = None
    for t in range(1, k + 1):
        pos = jnp.sum((cnt < t).astype(jnp.int32), axis=1, keepdims=True)
        if t == 1:
            pos1 = pos
        val = jnp.where(total >= t, pos, pos1)
        acc = acc + jnp.where(kiota == (t - 1), val, 0)
    o_ref[0] = acc


def _run_ball(px, py, pz, cx, cy, cz, radius, k, N, S, TS):
    r2 = radius ** 2
    grid = (_B, S // TS)
    pspec = pl.BlockSpec((1, 1, N), lambda b, t: (b, 0, 0))
    cspec = pl.BlockSpec((1, TS, 1), lambda b, t: (b, t, 0))
    return pl.pallas_call(
        functools.partial(_ball_kernel, r2=r2, k=k, N=N, TS=TS),
        grid=grid,
        in_specs=[pspec, pspec, pspec, cspec, cspec, cspec],
        out_specs=pl.BlockSpec((1, TS, k), lambda b, t: (b, t, 0)),
        out_shape=jax.ShapeDtypeStruct((_B, S, k), jnp.int32),
    )(px, py, pz, cx, cy, cz)


def _sc_gather(table, idx_flat, *, N, rows_batch, D):
    R = idx_flat.shape[0]
    NW = 32
    CH = 128
    rows_w = R // NW
    nch = rows_w // CH
    mesh = plsc.VectorSubcoreMesh(core_axis_name="c", subcore_axis_name="s")

    @functools.partial(
        pl.kernel, mesh=mesh,
        out_type=jax.ShapeDtypeStruct((R, D), jnp.float32),
        compiler_params=pltpu.CompilerParams(use_tc_tiling_on_sc=False),
        scratch_types=[
            pltpu.VMEM((CH,), jnp.int32),
            pltpu.VMEM((CH, D), jnp.float32),
            pltpu.SemaphoreType.DMA,
        ],
    )
    def gather_fn(table_hbm, idx_hbm, out_hbm, idx_v, rows_v, sem):
        wid = lax.axis_index("s") * 2 + lax.axis_index("c")
        base = wid * rows_w

        def body(i, carry):
            g0 = base + i * CH
            pltpu.sync_copy(idx_hbm.at[pl.ds(g0, CH)], idx_v)
            off = (g0 // rows_batch) * N
            for j in range(CH // 16):
                sl = pl.ds(j * 16, 16)
                idx_v[sl] = idx_v[sl] + off
            pltpu.async_copy(table_hbm.at[idx_v], rows_v, sem).wait()
            pltpu.sync_copy(rows_v, out_hbm.at[pl.ds(g0, CH)])
            return carry

        lax.fori_loop(0, nch, body, 0)

    return gather_fn(table, idx_flat)


def _p1_kernel(g_ref, c_ref, w_ref, b_ref, y_ref, s1_ref, s2_ref, *, TS, k):
    D = g_ref.shape[1]
    g = g_ref[...]
    c = c_ref[...]
    g = (g.reshape(TS, k, D) - c[:, None, :]).reshape(TS * k, D)
    y = jnp.dot(g, w_ref[...], preferred_element_type=jnp.float32) + b_ref[...]
    y_ref[...] = y
    s1_ref[0] = jnp.sum(y, axis=0, keepdims=True)
    s2_ref[0] = jnp.sum(y * y, axis=0, keepdims=True)


def _mm_kernel(g_ref, w_ref, b_ref, y_ref, s1_ref, s2_ref):
    y = jnp.dot(g_ref[...], w_ref[...],
                preferred_element_type=jnp.float32) + b_ref[...]
    y_ref[...] = y
    s1_ref[0] = jnp.sum(y, axis=0, keepdims=True)
    s2_ref[0] = jnp.sum(y * y, axis=0, keepdims=True)


def _p2_kernel(y_ref, sc_ref, sh_ref, w_ref, b_ref, o_ref, s1_ref, s2_ref):
    h = jnp.maximum(y_ref[...] * sc_ref[...] + sh_ref[...], 0.0)
    y = jnp.dot(h, w_ref[...], preferred_element_type=jnp.float32) + b_ref[...]
    o_ref[...] = y
    s1_ref[0] = jnp.sum(y, axis=0, keepdims=True)
    s2_ref[0] = jnp.sum(y * y, axis=0, keepdims=True)


def _fold_kernel(s1_ref, s2_ref, gm_ref, bt_ref, sc_ref, sh_ref, *, inv_m):
    s1 = jnp.sum(s1_ref[...], axis=0)
    s2 = jnp.sum(s2_ref[...], axis=0)
    mean = s1 * inv_m
    var = s2 * inv_m - mean * mean
    sc = gm_ref[...] * lax.rsqrt(var + 1e-5)
    sc_ref[...] = sc
    sh_ref[...] = bt_ref[...] - mean * sc


def _pool_kernel(y_ref, sc_ref, sh_ref, c_ref, o_ref, *, TS, k):
    C = y_ref.shape[1]
    h = jnp.maximum(y_ref[...] * sc_ref[...] + sh_ref[...], 0.0)
    p = jnp.max(h.reshape(TS, k, C), axis=1)
    o_ref[...] = jnp.concatenate([c_ref[:, :_PAD], p], axis=1)


def _pool4_kernel(y_ref, sc_ref, sh_ref, o_ref):
    C = y_ref.shape[1]
    h = jnp.maximum(y_ref[...] * sc_ref[...] + sh_ref[...], 0.0)
    o_ref[...] = jnp.max(h.reshape(_B, h.shape[0] // _B, C), axis=1)


def _full_spec(shape):
    nd = len(shape)
    return pl.BlockSpec(shape, lambda *a: (0,) * nd)


def _run_matmul_pass(kern, ins, full_ins, R, C, TR):
    G = R // TR
    in_specs = [pl.BlockSpec((TR, a.shape[1]), lambda i: (i, 0)) for a in ins]
    in_specs += [_full_spec(a.shape) for a in full_ins]
    out_shape = [
        jax.ShapeDtypeStruct((R, C), jnp.float32),
        jax.ShapeDtypeStruct((G, 1, C), jnp.float32),
        jax.ShapeDtypeStruct((G, 1, C), jnp.float32),
    ]
    out_specs = [
        pl.BlockSpec((TR, C), lambda i: (i, 0)),
        pl.BlockSpec((1, 1, C), lambda i: (i, 0, 0)),
        pl.BlockSpec((1, 1, C), lambda i: (i, 0, 0)),
    ]
    return pl.pallas_call(
        kern, grid=(G,), in_specs=in_specs, out_specs=out_specs,
        out_shape=out_shape,
    )(*ins, *full_ins)


def _run_p1(g, centers, w, b, R, C, TS, k):
    G = R // (TS * k)
    D = g.shape[1]
    in_specs = [
        pl.BlockSpec((TS * k, D), lambda i: (i, 0)),
        pl.BlockSpec((TS, D), lambda i: (i, 0)),
        _full_spec(w.shape),
        _full_spec(b.shape),
    ]
    out_shape = [
        jax.ShapeDtypeStruct((R, C), jnp.float32),
        jax.ShapeDtypeStruct((G, 1, C), jnp.float32),
        jax.ShapeDtypeStruct((G, 1, C), jnp.float32),
    ]
    out_specs = [
        pl.BlockSpec((TS * k, C), lambda i: (i, 0)),
        pl.BlockSpec((1, 1, C), lambda i: (i, 0, 0)),
        pl.BlockSpec((1, 1, C), lambda i: (i, 0, 0)),
    ]
    return pl.pallas_call(
        functools.partial(_p1_kernel, TS=TS, k=k),
        grid=(G,), in_specs=in_specs, out_specs=out_specs,
        out_shape=out_shape,
    )(g, centers, w, b)


def _run_fold(s1, s2, gamma, beta, inv_m):
    C = gamma.shape[1]
    return pl.pallas_call(
        functools.partial(_fold_kernel, inv_m=inv_m),
        out_shape=[jax.ShapeDtypeStruct((1, C), jnp.float32)] * 2,
    )(s1, s2, gamma, beta)


def _run_pool(y, sc, sh, centers, R, TS, k):
    C = y.shape[1]
    G = R // (TS * k)
    Dout = _PAD + C
    in_specs = [
        pl.BlockSpec((TS * k, C), lambda i: (i, 0)),
        _full_spec(sc.shape),
        _full_spec(sh.shape),
        pl.BlockSpec((TS, centers.shape[1]), lambda i: (i, 0)),
    ]
    return pl.pallas_call(
        functools.partial(_pool_kernel, TS=TS, k=k),
        grid=(G,), in_specs=in_specs,
        out_specs=pl.BlockSpec((TS, Dout), lambda i: (i, 0)),
        out_shape=jax.ShapeDtypeStruct((R // k, Dout), jnp.float32),
    )(y, sc, sh, centers)


def _prep_w1(layer, cin):
    W = layer['W']
    C = W.shape[0]
    D = _PAD + cin if cin else _PAD
    wp = jnp.zeros((D, C), jnp.float32)
    wp = wp.at[0:3, :].set(W[:, 0:3].T)
    if cin:
        wp = wp.at[_PAD:, :].set(W[:, 3:].T)
    return wp, layer['b'].reshape(1, C)


def _prep_w(layer):
    W = layer['W']
    return W.T, layer['b'].reshape(1, W.shape[0])


def _gb(layer):
    C = layer['W'].shape[0]
    return layer['gamma'].reshape(1, C), layer['beta'].reshape(1, C)


def _sa_mlp(g, centers, layers, R, TS, k, cin, group_all):
    w1, b1 = _prep_w1(layers[0], cin)
    C1 = w1.shape[1]
    if group_all:
        y, s1, s2 = _run_matmul_pass(_mm_kernel, [g], [w1, b1], R, C1, TS * k)
    else:
        y, s1, s2 = _run_p1(g, centers, w1, b1, R, C1, TS, k)
    gm, bt = _gb(layers[0])
    sc, sh = _run_fold(s1, s2, gm, bt, 1.0 / R)
    for layer in layers[1:]:
        w, b = _prep_w(layer)
        C = w.shape[1]
        y, s1, s2 = _run_matmul_pass(
            _p2_kernel, [y], [sc, sh, w, b], R, C, TS * k)
        gm, bt = _gb(layer)
        sc, sh = _run_fold(s1, s2, gm, bt, 1.0 / R)
    return y, sc, sh


def kernel(x, params):
    B, N0 = _B, _N0
    xs = x[:, 0, :]
    ys = x[:, 1, :]
    zs = x[:, 2, :]

    (cx1, cy1, cz1, cx2, cy2, cz2, cx3, cy3, cz3) = _run_fps(xs, ys, zs)

    def _p3d(a):
        return a.reshape(B, 1, a.shape[1])

    def _c3d(a):
        return a.reshape(B, a.shape[1], 1)

    idx1 = _run_ball(_p3d(xs), _p3d(ys), _p3d(zs),
                     _c3d(cx1), _c3d(cy1), _c3d(cz1),
                     _RADII[0], _NSAMPLES[0], N0, _NPOINTS[0], 256)
    idx2 = _run_ball(_p3d(cx1), _p3d(cy1), _p3d(cz1),
                     _c3d(cx2), _c3d(cy2), _c3d(cz2),
                     _RADII[1], _NSAMPLES[1], _NPOINTS[0], _NPOINTS[1], 256)
    idx3 = _run_ball(_p3d(cx2), _p3d(cy2), _p3d(cz2),
                     _c3d(cx3), _c3d(cy3), _c3d(cz3),
                     _RADII[2], _NSAMPLES[2], _NPOINTS[1], _NPOINTS[2], 128)

    aug0 = jnp.zeros((B * N0, _PAD), jnp.float32)
    xyz0 = jnp.stack([xs, ys, zs], axis=-1).reshape(B * N0, 3)
    aug0 = aug0.at[:, 0:3].set(xyz0)

    def centers_pad(cx, cy, cz, D):
        S = cx.shape[1]
        cp = jnp.zeros((B * S, D), jnp.float32)
        c3 = jnp.stack([cx, cy, cz], axis=-1).reshape(B * S, 3)
        return cp.at[:, 0:3].set(c3)

    cp1 = centers_pad(cx1, cy1, cz1, _PAD)
    cp2 = centers_pad(cx2, cy2, cz2, _PAD + 128)
    cp3 = centers_pad(cx3, cy3, cz3, _PAD + 256)

    S1, k1 = _NPOINTS[0], _NSAMPLES[0]
    R1 = B * S1 * k1
    g1 = _sc_gather(aug0, idx1.reshape(R1), N=N0, rows_batch=S1 * k1, D=_PAD)
    y1, sc1, sh1 = _sa_mlp(g1, cp1, params[0], R1, 256, k1, 0, False)
    aug1 = _run_pool(y1, sc1, sh1, cp1, R1, 256, k1)

    S2, k2 = _NPOINTS[1], _NSAMPLES[1]
    R2 = B * S2 * k2
    g2 = _sc_gather(aug1, idx2.reshape(R2), N=S1, rows_batch=S2 * k2,
                    D=_PAD + 128)
    y2, sc2, sh2 = _sa_mlp(g2, cp2, params[1], R2, 256, k2, 128, False)
    aug2 = _run_pool(y2, sc2, sh2, cp2, R2, 256, k2)

    S3, k3 = _NPOINTS[2], _NSAMPLES[2]
    R3 = B * S3 * k3
    g3 = _sc_gather(aug2, idx3.reshape(R3), N=S2, rows_batch=S3 * k3,
                    D=_PAD + 256)
    y3, sc3, sh3 = _sa_mlp(g3, cp3, params[2], R3, 128, k3, 256, False)
    aug3 = _run_pool(y3, sc3, sh3, cp3, R3, 128, k3)

    R4 = B * S3
    y4, sc4, sh4 = _sa_mlp(aug3, None, params[3], R4, 8, S3, 256, True)
    out = pl.pallas_call(
        _pool4_kernel,
        in_specs=[_full_spec(y4.shape), _full_spec(sc4.shape),
                  _full_spec(sh4.shape)],
        out_specs=_full_spec((B, y4.shape[1])),
        out_shape=jax.ShapeDtypeStruct((B, y4.shape[1]), jnp.float32),
    )(y4, sc4, sh4)
    return out
